# baseline (speedup 1.0000x reference)
import dataclasses
import functools

import jax
import jax.numpy as jnp
import numpy as np
from jax import lax
from jax.experimental import pallas as pl
from jax.experimental.pallas import tpu as pltpu
from jax.experimental.pallas import tpu_sc as plsc

ALPHA = 0.1
K_HOPS = 10

NC = 2
NS = 16
LANES = 16
EB = 128
CH = 104


def _sc_hop(h2, src4, dst3, val3, zeros, x2p, n_nodes, dh, nb):
    rows_main = (n_nodes // NS) & ~7
    rem = n_nodes - rows_main * NS
    n_ch = rows_main // CH
    assert n_ch * CH == rows_main and CH <= EB and rem <= EB

    mesh = plsc.VectorSubcoreMesh(core_axis_name="c", subcore_axis_name="s")

    cp = pltpu.CompilerParams()
    fields = pltpu.CompilerParams.__dataclass_fields__
    if "needs_layout_passes" in fields:
        cp = dataclasses.replace(cp, needs_layout_passes=False)
    if "use_tc_tiling_on_sc" in fields:
        cp = dataclasses.replace(cp, use_tc_tiling_on_sc=False)

    @functools.partial(
        pl.kernel,
        out_type=jax.ShapeDtypeStruct((NC * n_nodes, dh), jnp.bfloat16),
        mesh=mesh,
        compiler_params=cp,
        scratch_types=[
            pltpu.VMEM((nb, EB), jnp.int32),
            pltpu.VMEM((nb, EB), jnp.int32),
            pltpu.VMEM((nb, EB), jnp.float32),
            pltpu.VMEM((2, EB, dh), jnp.bfloat16),
            pltpu.VMEM((EB, dh), jnp.float32),
            pltpu.VMEM((CH, dh), jnp.float32),
            pltpu.VMEM_SHARED((n_nodes, dh), jnp.float32),
            pltpu.SemaphoreType.DMA,
            pltpu.SemaphoreType.DMA,
            pltpu.SemaphoreType.DMA,
        ],
    )
    def prop(h_hbm, src_hbm, dst_hbm, val_hbm, zero_hbm, x2_hbm, out_hbm,
             src_all, dst_all, val_all, rows_v, scat_v, xv, acc_sh,
             sem_i, sem_g0, sem_g1):
        cid = lax.axis_index("c")
        sid = lax.axis_index("s")
        wid = cid * NS + sid
        sem_g = (sem_g0, sem_g1)

        pltpu.async_copy(src_hbm.at[wid], src_all, sem_i)
        pltpu.async_copy(dst_hbm.at[sid], dst_all, sem_i)
        pltpu.async_copy(val_hbm.at[sid], val_all, sem_i)

        r0 = sid * rows_main
        pltpu.sync_copy(zero_hbm.at[pl.ds(r0, rows_main)],
                        acc_sh.at[pl.ds(r0, rows_main)])
        if rem:
            @pl.when(sid == NS - 1)
            def _():
                pltpu.sync_copy(zero_hbm.at[pl.ds(rows_main * NS, rem)],
                                acc_sh.at[pl.ds(rows_main * NS, rem)])

        pltpu.make_async_copy(src_hbm.at[wid], src_all, sem_i).wait()
        pltpu.make_async_copy(dst_hbm.at[sid], dst_all, sem_i).wait()
        pltpu.make_async_copy(val_hbm.at[sid], val_all, sem_i).wait()

        pltpu.async_copy(h_hbm.at[src_all.at[0]], rows_v.at[0], sem_g0)

        plsc.subcore_barrier()

        def substep(k, p):
            q = 1 - p
            pltpu.make_async_copy(
                h_hbm.at[src_all.at[k]], rows_v.at[p], sem_g[p]).wait()

            @pl.when(k + 1 < nb)
            def _():
                pltpu.async_copy(
                    h_hbm.at[src_all.at[k + 1]], rows_v.at[q], sem_g[q])

            @pl.loop(0, EB)
            def _(r):
                vv = plsc.load_gather(
                    val_all, [jnp.full((LANES,), k, dtype=jnp.int32),
                              jnp.full((LANES,), r, dtype=jnp.int32)])
                for g in range(dh // (2 * LANES)):
                    chunk = rows_v[p, r, pl.ds(g * 2 * LANES, 2 * LANES)]
                    a, b = plsc.unpack(chunk,
                                       format=plsc.PackFormat.INTERLEAVED)
                    base = g * 2 * LANES
                    scat_v[r, pl.ds(base, LANES)] = a * vv
                    scat_v[r, pl.ds(base + LANES, LANES)] = b * vv

            pltpu.sync_copy(scat_v, acc_sh.at[dst_all.at[k]], add=True)

        @pl.loop(0, nb // 2)
        def _(i):
            substep(2 * i, 0)
            substep(2 * i + 1, 1)

        plsc.subcore_barrier()

        def combine_rows(row0, nrows):
            a_v = scat_v.at[pl.ds(0, nrows)]
            x_v = xv.at[pl.ds(0, nrows)]
            pltpu.sync_copy(acc_sh.at[pl.ds(row0, nrows)], a_v)
            pltpu.sync_copy(x2_hbm.at[pl.ds(cid * n_nodes + row0, nrows)], x_v)

            @pl.loop(0, nrows)
            def _(r):
                for g in range(dh // (2 * LANES)):
                    base = g * 2 * LANES
                    ha = ((1.0 - ALPHA) * scat_v[r, pl.ds(base, LANES)]
                          + ALPHA * xv[r, pl.ds(base, LANES)])
                    hb = ((1.0 - ALPHA) * scat_v[r, pl.ds(base + LANES, LANES)]
                          + ALPHA * xv[r, pl.ds(base + LANES, LANES)])
                    rows_v[0, r, pl.ds(base, 2 * LANES)] = plsc.pack(
                        ha, hb, format=plsc.PackFormat.INTERLEAVED)

            pltpu.sync_copy(
                rows_v.at[0, pl.ds(0, nrows)],
                out_hbm.at[pl.ds(cid * n_nodes + row0, nrows)])

        @pl.loop(0, n_ch)
        def _(j):
            combine_rows(r0 + j * CH, CH)

        if rem:
            @pl.when(sid == NS - 1)
            def _():
                combine_rows(rows_main * NS, rem)

    return prop(h2, src4, dst3, val3, zeros, x2p)


def kernel(x, edge_index, adj_values):
    n_nodes, d = x.shape
    dh = d // NC
    dst = edge_index[0]
    src = edge_index[1]
    e = dst.shape[0]

    nb = -(-e // (NS * EB))
    nb += nb % 2
    e_pad = nb * EB * NS
    pad = e_pad - e
    if pad:
        src = jnp.concatenate([src, jnp.zeros((pad,), src.dtype)])
        dst = jnp.concatenate([dst, jnp.zeros((pad,), dst.dtype)])
        adj = jnp.concatenate([adj_values, jnp.zeros((pad,), adj_values.dtype)])
    else:
        adj = adj_values
    src3 = src.reshape(NS, nb, EB)
    src4 = jnp.concatenate([src3, src3 + n_nodes], axis=0)
    dst3 = dst.reshape(NS, nb, EB)
    val3 = adj.reshape(NS, nb, EB)
    zeros = jnp.zeros((n_nodes, dh), jnp.float32)

    x2 = jnp.concatenate([x[:, :dh], x[:, dh:]], axis=0)
    perm = np.concatenate(
        [np.concatenate([np.arange(g * 32, (g + 1) * 32, 2),
                         np.arange(g * 32 + 1, (g + 1) * 32, 2)])
         for g in range(dh // 32)])
    x2p = x2[:, perm]

    h2 = x2.astype(jnp.bfloat16)
    for _ in range(K_HOPS):
        h2 = _sc_hop(h2, src4, dst3, val3, zeros, x2p, n_nodes, dh, nb)

    h2 = h2.astype(jnp.float32)
    return jnp.concatenate([h2[:n_nodes], h2[n_nodes:]], axis=1)


# speedup vs baseline: 1.1214x; 1.1214x over previous
import dataclasses
import functools

import jax
import jax.numpy as jnp
from jax import lax
from jax.experimental import pallas as pl
from jax.experimental.pallas import tpu as pltpu
from jax.experimental.pallas import tpu_sc as plsc

ALPHA = 0.1
K_HOPS = 10

NC = 2
NS = 16
NW = NC * NS
LANES = 16
EB = 64
CH = 104


def _sc_propagate(h, src3, dst3, val3, zeros, n_nodes, d, nb):
    rows_main = (n_nodes // NS) & ~7
    rem = n_nodes - rows_main * NS

    mesh = plsc.VectorSubcoreMesh(core_axis_name="c", subcore_axis_name="s")

    cp = pltpu.CompilerParams()
    fields = pltpu.CompilerParams.__dataclass_fields__
    if "needs_layout_passes" in fields:
        cp = dataclasses.replace(cp, needs_layout_passes=False)
    if "use_tc_tiling_on_sc" in fields:
        cp = dataclasses.replace(cp, use_tc_tiling_on_sc=False)

    @functools.partial(
        pl.kernel,
        out_type=jax.ShapeDtypeStruct((NC * n_nodes, d), jnp.float32),
        mesh=mesh,
        compiler_params=cp,
        scratch_types=[
            pltpu.VMEM((nb, EB), jnp.int32),
            pltpu.VMEM((nb, EB), jnp.int32),
            pltpu.VMEM((nb, EB), jnp.float32),
            pltpu.VMEM((2, EB, d), jnp.float32),
            pltpu.VMEM_SHARED((n_nodes, d), jnp.float32),
            pltpu.SemaphoreType.DMA,
            pltpu.SemaphoreType.DMA,
            pltpu.SemaphoreType.DMA,
        ],
    )
    def prop(h_hbm, src_hbm, dst_hbm, val_hbm, zero_hbm, out_hbm,
             src_all, dst_all, val_all, rows_v, acc_sh, sem_i, sem_g0, sem_g1):
        cid = lax.axis_index("c")
        sid = lax.axis_index("s")
        wid = cid * NS + sid
        sem_g = (sem_g0, sem_g1)

        pltpu.async_copy(src_hbm.at[wid], src_all, sem_i)
        pltpu.async_copy(dst_hbm.at[wid], dst_all, sem_i)
        pltpu.async_copy(val_hbm.at[wid], val_all, sem_i)

        r0 = sid * rows_main
        pltpu.sync_copy(zero_hbm.at[pl.ds(r0, rows_main)],
                        acc_sh.at[pl.ds(r0, rows_main)])
        if rem:
            @pl.when(sid == NS - 1)
            def _():
                pltpu.sync_copy(zero_hbm.at[pl.ds(rows_main * NS, rem)],
                                acc_sh.at[pl.ds(rows_main * NS, rem)])

        pltpu.make_async_copy(src_hbm.at[wid], src_all, sem_i).wait()
        pltpu.make_async_copy(dst_hbm.at[wid], dst_all, sem_i).wait()
        pltpu.make_async_copy(val_hbm.at[wid], val_all, sem_i).wait()

        pltpu.async_copy(h_hbm.at[src_all.at[0]], rows_v.at[0], sem_g0)

        plsc.subcore_barrier()

        def substep(k, p):
            q = 1 - p
            pltpu.make_async_copy(
                h_hbm.at[src_all.at[k]], rows_v.at[p], sem_g[p]).wait()

            @pl.when(k + 1 < nb)
            def _():
                pltpu.async_copy(
                    h_hbm.at[src_all.at[k + 1]], rows_v.at[q], sem_g[q])

            @pl.loop(0, EB)
            def _(r):
                vv = plsc.load_gather(
                    val_all, [jnp.full((LANES,), k, dtype=jnp.int32),
                              jnp.full((LANES,), r, dtype=jnp.int32)])
                for c in range(d // LANES):
                    sl = pl.ds(c * LANES, LANES)
                    rows_v[p, r, sl] = rows_v[p, r, sl] * vv

            pltpu.sync_copy(rows_v.at[p], acc_sh.at[dst_all.at[k]], add=True)

        @pl.loop(0, nb // 2)
        def _(i):
            substep(2 * i, 0)
            substep(2 * i + 1, 1)

        plsc.subcore_barrier()

        o0 = cid * n_nodes + r0
        pltpu.sync_copy(acc_sh.at[pl.ds(r0, rows_main)],
                        out_hbm.at[pl.ds(o0, rows_main)])
        if rem:
            @pl.when(sid == NS - 1)
            def _():
                pltpu.sync_copy(
                    acc_sh.at[pl.ds(rows_main * NS, rem)],
                    out_hbm.at[pl.ds(cid * n_nodes + rows_main * NS, rem)])

    return prop(h, src3, dst3, val3, zeros)


def _tc_combine(p, x, n_nodes, d):
    def body(p_ref, x_ref, o_ref):
        agg = p_ref[0:n_nodes, :] + p_ref[n_nodes:2 * n_nodes, :]
        o_ref[...] = (1.0 - ALPHA) * agg + ALPHA * x_ref[...]

    return pl.pallas_call(
        body,
        out_shape=jax.ShapeDtypeStruct((n_nodes, d), jnp.float32),
    )(p, x)


def kernel(x, edge_index, adj_values):
    n_nodes, d = x.shape
    dst = edge_index[0]
    src = edge_index[1]
    e = dst.shape[0]

    adj = adj_values
    nb = -(-e // (NW * EB))
    nb += nb % 2
    e_pad = nb * EB * NW
    pad = e_pad - e
    if pad:
        src = jnp.concatenate([src, jnp.zeros((pad,), src.dtype)])
        dst = jnp.concatenate([dst, jnp.zeros((pad,), dst.dtype)])
        adj = jnp.concatenate([adj, jnp.zeros((pad,), adj.dtype)])
    src3 = src.reshape(NW, nb, EB)
    dst3 = dst.reshape(NW, nb, EB)
    val3 = adj.reshape(NW, nb, EB)
    zeros = jnp.zeros((n_nodes, d), jnp.float32)

    h = x
    for _ in range(K_HOPS):
        p = _sc_propagate(h, src3, dst3, val3, zeros, n_nodes, d, nb)
        h = _tc_combine(p, x, n_nodes, d)
    return h


# speedup vs baseline: 1.5125x; 1.3488x over previous
import dataclasses
import functools

import jax
import jax.numpy as jnp
from jax import lax
from jax.experimental import pallas as pl
from jax.experimental.pallas import tpu as pltpu
from jax.experimental.pallas import tpu_sc as plsc

ALPHA = 0.1
K_HOPS = 10

NC = 2
NS = 16
LANES = 16
EB = 128
SB = 40
CH = 104


def _sc_appnp(x2, src3, dst3, val3, zeros, n_nodes, dh, nb):
    rows_main = (n_nodes // NS) & ~7
    rem = n_nodes - rows_main * NS
    n_ch = rows_main // CH
    nsg = nb // SB
    assert n_ch * CH == rows_main and CH <= EB and rem <= EB
    assert nsg * SB == nb and nsg % 2 == 0

    mesh = plsc.VectorSubcoreMesh(core_axis_name="c", subcore_axis_name="s")

    cp = pltpu.CompilerParams()
    fields = pltpu.CompilerParams.__dataclass_fields__
    if "needs_layout_passes" in fields:
        cp = dataclasses.replace(cp, needs_layout_passes=False)
    if "use_tc_tiling_on_sc" in fields:
        cp = dataclasses.replace(cp, use_tc_tiling_on_sc=False)

    @functools.partial(
        pl.kernel,
        out_type=jax.ShapeDtypeStruct((NC * n_nodes, dh), jnp.float32),
        mesh=mesh,
        compiler_params=cp,
        scratch_types=[
            pltpu.VMEM((2, SB, EB), jnp.int32),
            pltpu.VMEM((2, SB, EB), jnp.int32),
            pltpu.VMEM((2, SB, EB), jnp.float32),
            pltpu.VMEM((2, EB, dh), jnp.float32),
            pltpu.VMEM_SHARED((n_nodes, dh), jnp.float32),
            pltpu.VMEM_SHARED((n_nodes, dh), jnp.float32),
            pltpu.SemaphoreType.DMA,
            pltpu.SemaphoreType.DMA,
            pltpu.SemaphoreType.DMA,
            pltpu.SemaphoreType.DMA,
            pltpu.SemaphoreType.DMA,
            pltpu.SemaphoreType.DMA,
        ],
    )
    def prop(x2_hbm, src_hbm, dst_hbm, val_hbm, zero_hbm, out_hbm,
             seg_src, seg_dst, seg_val, rows_v, h_sh, acc_sh,
             sem_s0, sem_s1, sem_d0, sem_d1, sem_g0, sem_g1):
        cid = lax.axis_index("c")
        sid = lax.axis_index("s")
        sem_s = (sem_s0, sem_s1)
        sem_d = (sem_d0, sem_d1)
        sem_g = (sem_g0, sem_g1)
        r0 = sid * rows_main

        def src_copy(s, sp):
            return pltpu.make_async_copy(
                src_hbm.at[sid, pl.ds(s * SB, SB)], seg_src.at[sp], sem_s[sp])

        def dst_copy(s, sp):
            return pltpu.make_async_copy(
                dst_hbm.at[sid, pl.ds(s * SB, SB)], seg_dst.at[sp], sem_d[sp])

        def val_copy(s, sp):
            return pltpu.make_async_copy(
                val_hbm.at[sid, pl.ds(s * SB, SB)], seg_val.at[sp], sem_d[sp])

        def stage_seg(s, sp):
            src_copy(s, sp).start()
            dst_copy(s, sp).start()
            val_copy(s, sp).start()

        pltpu.sync_copy(x2_hbm.at[pl.ds(cid * n_nodes + r0, rows_main)],
                        h_sh.at[pl.ds(r0, rows_main)])
        pltpu.sync_copy(zero_hbm.at[pl.ds(r0, rows_main)],
                        acc_sh.at[pl.ds(r0, rows_main)])
        if rem:
            @pl.when(sid == NS - 1)
            def _():
                t0 = rows_main * NS
                pltpu.sync_copy(x2_hbm.at[pl.ds(cid * n_nodes + t0, rem)],
                                h_sh.at[pl.ds(t0, rem)])
                pltpu.sync_copy(zero_hbm.at[pl.ds(t0, rem)],
                                acc_sh.at[pl.ds(t0, rem)])

        plsc.subcore_barrier()

        def substep(s, sp, j, p):
            q = 1 - p
            pltpu.make_async_copy(
                h_sh.at[seg_src.at[sp, j]], rows_v.at[p], sem_g[p]).wait()

            @pl.when(j + 1 < SB)
            def _():
                pltpu.async_copy(
                    h_sh.at[seg_src.at[sp, j + 1]], rows_v.at[q], sem_g[q])

            @pl.when(j + 1 == SB)
            def _():
                @pl.when(s + 1 < nsg)
                def _():
                    src_copy(s + 1, 1 - sp).wait()
                    pltpu.async_copy(
                        h_sh.at[seg_src.at[1 - sp, 0]], rows_v.at[q],
                        sem_g[q])

            @pl.loop(0, EB)
            def _(r):
                vv = plsc.load_gather(
                    seg_val, [jnp.full((LANES,), sp, dtype=jnp.int32),
                              jnp.full((LANES,), j, dtype=jnp.int32),
                              jnp.full((LANES,), r, dtype=jnp.int32)])
                for c in range(dh // LANES):
                    sl = pl.ds(c * LANES, LANES)
                    rows_v[p, r, sl] = rows_v[p, r, sl] * vv

            pltpu.sync_copy(rows_v.at[p], acc_sh.at[seg_dst.at[sp, j]],
                            add=True)

        def seg_body(s, sp):
            dst_copy(s, sp).wait()
            val_copy(s, sp).wait()

            @pl.loop(0, SB // 2)
            def _(jh):
                substep(s, sp, 2 * jh, 0)
                substep(s, sp, 2 * jh + 1, 1)

            @pl.when(s + 2 < nsg)
            def _():
                stage_seg(s + 2, sp)

        def combine_rows(row0, nrows):
            a_v = rows_v.at[0, pl.ds(0, nrows)]
            x_v = rows_v.at[1, pl.ds(0, nrows)]
            pltpu.sync_copy(acc_sh.at[pl.ds(row0, nrows)], a_v)
            pltpu.sync_copy(x2_hbm.at[pl.ds(cid * n_nodes + row0, nrows)], x_v)
            pltpu.sync_copy(zero_hbm.at[pl.ds(row0, nrows)],
                            acc_sh.at[pl.ds(row0, nrows)])

            @pl.loop(0, nrows)
            def _(r):
                for c in range(dh // LANES):
                    sl = pl.ds(c * LANES, LANES)
                    rows_v[0, r, sl] = ((1.0 - ALPHA) * rows_v[0, r, sl]
                                        + ALPHA * rows_v[1, r, sl])

            pltpu.sync_copy(a_v, h_sh.at[pl.ds(row0, nrows)])

        @pl.loop(0, K_HOPS)
        def _(t):
            stage_seg(0, 0)
            stage_seg(1, 1)
            src_copy(0, 0).wait()
            pltpu.async_copy(h_sh.at[seg_src.at[0, 0]], rows_v.at[0], sem_g0)

            @pl.loop(0, nsg // 2)
            def _(i):
                seg_body(2 * i, 0)
                seg_body(2 * i + 1, 1)

            plsc.subcore_barrier()

            @pl.loop(0, n_ch)
            def _(j):
                combine_rows(r0 + j * CH, CH)

            if rem:
                @pl.when(sid == NS - 1)
                def _():
                    combine_rows(rows_main * NS, rem)

            plsc.subcore_barrier()

        pltpu.sync_copy(h_sh.at[pl.ds(r0, rows_main)],
                        out_hbm.at[pl.ds(cid * n_nodes + r0, rows_main)])
        if rem:
            @pl.when(sid == NS - 1)
            def _():
                t0 = rows_main * NS
                pltpu.sync_copy(
                    h_sh.at[pl.ds(t0, rem)],
                    out_hbm.at[pl.ds(cid * n_nodes + t0, rem)])

    return prop(x2, src3, dst3, val3, zeros)


def kernel(x, edge_index, adj_values):
    n_nodes, d = x.shape
    dh = d // NC
    dst = edge_index[0]
    src = edge_index[1]
    e = dst.shape[0]

    nb = -(-e // (NS * EB))
    nb = -(-nb // (2 * SB)) * (2 * SB)
    e_pad = nb * EB * NS
    pad = e_pad - e
    if pad:
        src = jnp.concatenate([src, jnp.zeros((pad,), src.dtype)])
        dst = jnp.concatenate([dst, jnp.zeros((pad,), dst.dtype)])
        adj = jnp.concatenate([adj_values, jnp.zeros((pad,), adj_values.dtype)])
    else:
        adj = adj_values
    src3 = src.reshape(NS, nb, EB)
    dst3 = dst.reshape(NS, nb, EB)
    val3 = adj.reshape(NS, nb, EB)
    zeros = jnp.zeros((n_nodes, dh), jnp.float32)

    x2 = jnp.concatenate([x[:, :dh], x[:, dh:]], axis=0)

    h2 = _sc_appnp(x2, src3, dst3, val3, zeros, n_nodes, dh, nb)

    return jnp.concatenate([h2[:n_nodes], h2[n_nodes:]], axis=1)
